# baseline (speedup 1.0000x reference)
import dataclasses
import functools

import jax
import jax.numpy as jnp
from jax import lax
from jax.experimental import pallas as pl
from jax.experimental.pallas import tpu as pltpu
from jax.experimental.pallas import tpu_sc as plsc

H = 128
N_PAD = 10240
E_PAD = 327680
BLK_N = 1024
BLK_E = 4096

NC = 2
NS = 16
NW = NC * NS
EB = E_PAD // NW
CH = 128
NCH = EB // CH
NROWS = N_PAD // NS

_VMESH = plsc.VectorSubcoreMesh(core_axis_name="c", subcore_axis_name="s")

_SC_PARAMS = pltpu.CompilerParams()
if "needs_layout_passes" in pltpu.CompilerParams.__dataclass_fields__:
    _SC_PARAMS = dataclasses.replace(_SC_PARAMS, needs_layout_passes=False)


def _ln(x, g, b):
    m = jnp.mean(x, axis=-1, keepdims=True)
    v = jnp.mean((x - m) ** 2, axis=-1, keepdims=True)
    return (x - m) * jax.lax.rsqrt(v + 1e-5) * g + b


def _full(shape=(H, H)):
    return pl.BlockSpec(shape, lambda i: (0,) * len(shape))


def _node_encoder_body(nf, w1t, b1, g, be, w2t, b2, wat, wbt, nh_o, a_o, b_o):
    x = jnp.dot(nf[...], w1t[...], preferred_element_type=jnp.float32) + b1[...]
    x = jax.nn.relu(_ln(x, g[...], be[...]))
    nh = jnp.dot(x, w2t[...], preferred_element_type=jnp.float32) + b2[...]
    nh_o[...] = nh
    a_o[...] = jnp.dot(nh, wat[...], preferred_element_type=jnp.float32)
    b_o[...] = jnp.dot(nh, wbt[...], preferred_element_type=jnp.float32)


def _node_encoder(nf, p, wat, wbt):
    f32 = jnp.float32
    return pl.pallas_call(
        _node_encoder_body,
        grid=(N_PAD // BLK_N,),
        in_specs=[
            pl.BlockSpec((BLK_N, H), lambda i: (i, 0)),
            _full(), _full((1, H)), _full((1, H)), _full((1, H)),
            _full(), _full((1, H)), _full(), _full(),
        ],
        out_specs=[pl.BlockSpec((BLK_N, H), lambda i: (i, 0))] * 3,
        out_shape=[jax.ShapeDtypeStruct((N_PAD, H), f32)] * 3,
    )(nf, p['ne_W1'].T, p['ne_b1'][None], p['ne_g'][None], p['ne_be'][None],
      p['ne_W2'].T, p['ne_b2'][None], wat, wbt)


def _edge_enc_body(ef, w1t, b1, g, be, w2t, b2, eh_o):
    x = jnp.dot(ef[...], w1t[...], preferred_element_type=jnp.float32) + b1[...]
    x = jax.nn.relu(_ln(x, g[...], be[...]))
    eh_o[...] = jnp.dot(x, w2t[...], preferred_element_type=jnp.float32) + b2[...]


def _edge_enc(ef, p):
    return pl.pallas_call(
        _edge_enc_body,
        grid=(E_PAD // BLK_E,),
        in_specs=[pl.BlockSpec((BLK_E, H), lambda i: (i, 0)),
                  _full(), _full((1, H)), _full((1, H)), _full((1, H)),
                  _full(), _full((1, H))],
        out_specs=pl.BlockSpec((BLK_E, H), lambda i: (i, 0)),
        out_shape=jax.ShapeDtypeStruct((E_PAD, H), jnp.float32),
    )(ef, p['ee_W1'].T, p['ee_b1'][None], p['ee_g'][None],
      p['ee_be'][None], p['ee_W2'].T, p['ee_b2'][None])


def _mlp_body(gs, gd, eh, w1et, b1, g, be, w2t, b2, gw, gb, m_o):
    m1 = (gs[...] + gd[...]
          + jnp.dot(eh[...], w1et[...], preferred_element_type=jnp.float32)
          + b1[...])
    m = jax.nn.relu(_ln(m1, g[...], be[...]))
    m = jnp.dot(m, w2t[...], preferred_element_type=jnp.float32) + b2[...]
    gate = jax.nn.sigmoid(
        jnp.sum(eh[...] * gw[...], axis=1, keepdims=True) + gb[...])
    m_o[...] = m * gate


def _edge_mlp(i, gs, gd, eh, p):
    return pl.pallas_call(
        _mlp_body,
        grid=(E_PAD // BLK_E,),
        in_specs=[pl.BlockSpec((BLK_E, H), lambda j: (j, 0)),
                  pl.BlockSpec((BLK_E, H), lambda j: (j, 0)),
                  pl.BlockSpec((BLK_E, H), lambda j: (j, 0)),
                  _full(), _full((1, H)), _full((1, H)), _full((1, H)),
                  _full(), _full((1, H)), _full((1, H)), _full((1, 1))],
        out_specs=pl.BlockSpec((BLK_E, H), lambda j: (j, 0)),
        out_shape=jax.ShapeDtypeStruct((E_PAD, H), jnp.float32),
    )(gs, gd, eh, p[f'mp{i}_W1'][:, 2 * H:3 * H].T, p[f'mp{i}_b1'][None],
      p[f'mp{i}_g'][None], p[f'mp{i}_be'][None],
      p[f'mp{i}_W2'].T, p[f'mp{i}_b2'][None],
      p[f'mp{i}_gW'], p[f'mp{i}_gb'][None])


def _gru_body(has_upd, aggp, nh, wit, bi, wht, bh, wat, wbt, ncwt, ncb,
              nh_o, a_o, b_o, upd_o=None):
    agg = aggp[0] + aggp[1]
    gi = jnp.dot(agg, wit[...], preferred_element_type=jnp.float32) + bi[...]
    gh = jnp.dot(nh[...], wht[...], preferred_element_type=jnp.float32) + bh[...]
    r = jax.nn.sigmoid(gi[:, :H] + gh[:, :H])
    z = jax.nn.sigmoid(gi[:, H:2 * H] + gh[:, H:2 * H])
    n = jnp.tanh(gi[:, 2 * H:] + r * gh[:, 2 * H:])
    nh_new = (1.0 - z) * n + z * nh[...]
    nh_o[...] = nh_new
    a_o[...] = jnp.dot(nh_new, wat[...], preferred_element_type=jnp.float32)
    b_o[...] = jnp.dot(nh_new, wbt[...], preferred_element_type=jnp.float32)
    if has_upd:
        upd_o[...] = jnp.dot(nh_new, ncwt[...],
                             preferred_element_type=jnp.float32) + ncb[...]


def _gru(i, aggp, nh, p, wat, wbt, last):
    f32 = jnp.float32
    n_out = 4 if last else 3
    blk = pl.BlockSpec((BLK_N, H), lambda j: (j, 0))
    return pl.pallas_call(
        functools.partial(_gru_body, last),
        grid=(N_PAD // BLK_N,),
        in_specs=[pl.BlockSpec((2, BLK_N, H), lambda j: (0, j, 0)), blk,
                  _full((H, 3 * H)), _full((1, 3 * H)),
                  _full((H, 3 * H)), _full((1, 3 * H)),
                  _full(), _full(), _full(), _full((1, H))],
        out_specs=[blk] * n_out,
        out_shape=[jax.ShapeDtypeStruct((N_PAD, H), f32)] * n_out,
    )(aggp, nh, p[f'mp{i}_Wi'].T, p[f'mp{i}_bi'][None],
      p[f'mp{i}_Wh'].T, p[f'mp{i}_bh'][None], wat, wbt,
      p['nc_W'].T, p['nc_b'][None])


def _scorer_body(gs, gd, b1, w2, b2, s_o):
    es = jax.nn.relu(gs[...] + gd[...] + b1[...])
    s_o[...] = jax.nn.sigmoid(
        jnp.sum(es * w2[...], axis=1, keepdims=True) + b2[...])


def _scorer(gs, gd, p):
    return pl.pallas_call(
        _scorer_body,
        grid=(E_PAD // BLK_E,),
        in_specs=[pl.BlockSpec((BLK_E, H), lambda j: (j, 0)),
                  pl.BlockSpec((BLK_E, H), lambda j: (j, 0)),
                  _full((1, H)), _full((1, H)), _full((1, 1))],
        out_specs=pl.BlockSpec((BLK_E, 1), lambda j: (j, 0)),
        out_shape=jax.ShapeDtypeStruct((E_PAD, 1), jnp.float32),
    )(gs, gd, p['ep_b1'][None], p['ep_W2'], p['ep_b2'][None])


def _gather2(a, b, srcr, dstr):
    f32 = jnp.float32

    @functools.partial(
        pl.kernel,
        out_type=[jax.ShapeDtypeStruct((E_PAD, H), f32)] * 2,
        mesh=_VMESH,
        scratch_types=[
            pltpu.VMEM((NCH, CH), jnp.int32),
            pltpu.VMEM((NCH, CH), jnp.int32),
        ] + [pltpu.VMEM((CH, H), f32)] * 4
        + [pltpu.SemaphoreType.DMA] * 8,
    )
    def k(a_hbm, b_hbm, srcr_hbm, dstr_hbm, gs_hbm, gd_hbm,
          idxs_v, idxd_v, a0, a1, b0, b1,
          sa0, sa1, sb0, sb1, soa0, soa1, sob0, sob1):
        bufa = (a0, a1)
        bufb = (b0, b1)
        sa = (sa0, sa1)
        sb = (sb0, sb1)
        soa = (soa0, soa1)
        sob = (sob0, sob1)
        wid = lax.axis_index("s") * NC + lax.axis_index("c")
        base = wid * EB
        pltpu.sync_copy(srcr_hbm.at[wid], idxs_v)
        pltpu.sync_copy(dstr_hbm.at[wid], idxd_v)

        def issue_gather(j, c):
            pltpu.async_copy(a_hbm.at[idxs_v.at[c]], bufa[j], sa[j])
            pltpu.async_copy(b_hbm.at[idxd_v.at[c]], bufb[j], sb[j])

        def wait_gather(j):
            pltpu.make_async_copy(a_hbm.at[pl.ds(0, CH)], bufa[j], sa[j]).wait()
            pltpu.make_async_copy(b_hbm.at[pl.ds(0, CH)], bufb[j], sb[j]).wait()

        def issue_out(j, c):
            row = pl.ds(base + c * CH, CH)
            pltpu.async_copy(bufa[j], gs_hbm.at[row], soa[j])
            pltpu.async_copy(bufb[j], gd_hbm.at[row], sob[j])

        def wait_out(j):
            pltpu.make_async_copy(a_hbm.at[pl.ds(0, CH)], bufa[j],
                                  soa[j]).wait()
            pltpu.make_async_copy(a_hbm.at[pl.ds(0, CH)], bufb[j],
                                  sob[j]).wait()

        for j in (0, 1):
            issue_gather(j, j)
        for j in (0, 1):
            wait_gather(j)
            issue_out(j, j)

        @pl.loop(1, NCH // 2)
        def _(kk):
            for j in (0, 1):
                c = kk * 2 + j
                wait_out(j)
                issue_gather(j, c)
                wait_gather(j)
                issue_out(j, c)

        for j in (0, 1):
            wait_out(j)

    return k(a, b, srcr, dstr)


def _scatter_add(m, dstr):
    f32 = jnp.float32

    @functools.partial(
        pl.kernel,
        out_type=jax.ShapeDtypeStruct((NC, N_PAD, H), f32),
        mesh=_VMESH,
        scratch_types=[
            pltpu.VMEM_SHARED((N_PAD, H), f32),
            pltpu.VMEM((CH, H), f32),
            pltpu.VMEM((CH, H), f32),
            pltpu.VMEM((NCH, CH), jnp.int32),
            pltpu.SemaphoreType.DMA,
            pltpu.SemaphoreType.DMA,
        ],
    )
    def k(m_hbm, dstr_hbm, out_hbm, acc_sh, m0, m1, idx_v, sm0, sm1):
        bufm = (m0, m1)
        sm = (sm0, sm1)
        cid = lax.axis_index("c")
        sid = lax.axis_index("s")
        wid = sid * NC + cid
        base = wid * EB

        @pl.loop(0, CH)
        def _(r):
            for c in range(0, H, 16):
                m0.at[r, pl.ds(c, 16)][...] = jnp.zeros((16,), f32)

        @pl.loop(0, NROWS // CH)
        def _(jj):
            pltpu.sync_copy(m0, acc_sh.at[pl.ds(sid * NROWS + jj * CH, CH)])

        plsc.subcore_barrier()
        pltpu.sync_copy(dstr_hbm.at[wid], idx_v)

        def issue_m(j, c):
            pltpu.async_copy(m_hbm.at[pl.ds(base + c * CH, CH)], bufm[j], sm[j])

        def wait_m(j):
            pltpu.make_async_copy(m_hbm.at[pl.ds(0, CH)], bufm[j], sm[j]).wait()

        for j in (0, 1):
            issue_m(j, j)

        @pl.loop(0, NCH // 2)
        def _(kk):
            for j in (0, 1):
                c = kk * 2 + j
                wait_m(j)
                pltpu.sync_copy(bufm[j], acc_sh.at[idx_v.at[c]], add=True)

                @pl.when(c + 2 < NCH)
                def _():
                    issue_m(j, c + 2)

        plsc.subcore_barrier()
        pltpu.sync_copy(acc_sh.at[pl.ds(sid * NROWS, NROWS)],
                        out_hbm.at[cid, pl.ds(sid * NROWS, NROWS)])

    return k(m, dstr)


def _pack_bf16(x):
    return jax.lax.bitcast_convert_type(
        x.reshape(x.shape[0], HW, 2), jnp.int32)


def _unpack_bf16(x):
    return jax.lax.bitcast_convert_type(
        x, jnp.bfloat16).reshape(x.shape[0], H)


@jax.jit
def _run(node_features, edge_indices, edge_features, p):
    nf = jnp.zeros((N_PAD, H), jnp.float32).at[:node_features.shape[0]].set(
        node_features)
    ef = jnp.zeros((E_PAD, H), jnp.float32).at[:edge_features.shape[0]].set(
        edge_features)
    e = edge_features.shape[0]
    n = node_features.shape[0]
    src = jnp.full((E_PAD,), 0, jnp.int32).at[:e].set(edge_indices[0])
    dst = jnp.full((E_PAD,), n, jnp.int32).at[:e].set(edge_indices[1])
    src = src.reshape(NW, NCH, CH)
    dst = dst.reshape(NW, NCH, CH)

    sw = lambda i: p[f'mp{i}_W1'][:, :H].T
    dw = lambda i: p[f'mp{i}_W1'][:, H:2 * H].T

    nh, a, b = _node_encoder(nf, p, sw(0), dw(0))
    eh = _edge_enc(ef, p)

    for i in range(3):
        gs, gd = _gather2(a, b, src, dst)
        m = _edge_mlp(i, gs, gd, eh, p)
        aggp = _scatter_add(m, dst)
        last = i == 2
        if last:
            nwat, nwbt = p['ep_W1'][:, :H].T, p['ep_W1'][:, H:].T
        else:
            nwat, nwbt = sw(i + 1), dw(i + 1)
        outs = _gru(i, aggp, nh, p, nwat, nwbt, last)
        nh, a, b = outs[0], outs[1], outs[2]
        if last:
            upd = outs[3]

    gs, gd = _gather2(a, b, src, dst)
    scores = _scorer(gs, gd, p)
    return upd[:n], scores[:e]


def kernel(node_features, node_boxes, edge_indices, edge_features, params):
    del node_boxes
    return _run(node_features, edge_indices, edge_features, params)


# speedup vs baseline: 1.0251x; 1.0251x over previous
import dataclasses
import functools

import jax
import jax.numpy as jnp
from jax import lax
from jax.experimental import pallas as pl
from jax.experimental.pallas import tpu as pltpu
from jax.experimental.pallas import tpu_sc as plsc

H = 128
N_PAD = 10240
E_PAD = 327680
BLK_N = 1024
BLK_E = 4096

NC = 2
NS = 16
NW = NC * NS
EB = E_PAD // NW
CH = 128
NCH = EB // CH
GCH = 80
GNCH = EB // GCH
NROWS = N_PAD // NS

_VMESH = plsc.VectorSubcoreMesh(core_axis_name="c", subcore_axis_name="s")

_SC_PARAMS = pltpu.CompilerParams()
if "needs_layout_passes" in pltpu.CompilerParams.__dataclass_fields__:
    _SC_PARAMS = dataclasses.replace(_SC_PARAMS, needs_layout_passes=False)


def _ln(x, g, b):
    m = jnp.mean(x, axis=-1, keepdims=True)
    v = jnp.mean((x - m) ** 2, axis=-1, keepdims=True)
    return (x - m) * jax.lax.rsqrt(v + 1e-5) * g + b


def _full(shape=(H, H)):
    return pl.BlockSpec(shape, lambda i: (0,) * len(shape))


def _node_encoder_body(nf, w1t, b1, g, be, w2t, b2, wat, wbt, nh_o, a_o, b_o):
    x = jnp.dot(nf[...], w1t[...], preferred_element_type=jnp.float32) + b1[...]
    x = jax.nn.relu(_ln(x, g[...], be[...]))
    nh = jnp.dot(x, w2t[...], preferred_element_type=jnp.float32) + b2[...]
    nh_o[...] = nh
    a_o[...] = jnp.dot(nh, wat[...], preferred_element_type=jnp.float32)
    b_o[...] = jnp.dot(nh, wbt[...], preferred_element_type=jnp.float32)


def _node_encoder(nf, p, wat, wbt):
    f32 = jnp.float32
    return pl.pallas_call(
        _node_encoder_body,
        grid=(N_PAD // BLK_N,),
        in_specs=[
            pl.BlockSpec((BLK_N, H), lambda i: (i, 0)),
            _full(), _full((1, H)), _full((1, H)), _full((1, H)),
            _full(), _full((1, H)), _full(), _full(),
        ],
        out_specs=[pl.BlockSpec((BLK_N, H), lambda i: (i, 0))] * 3,
        out_shape=[jax.ShapeDtypeStruct((N_PAD, H), f32)] * 3,
    )(nf, p['ne_W1'].T, p['ne_b1'][None], p['ne_g'][None], p['ne_be'][None],
      p['ne_W2'].T, p['ne_b2'][None], wat, wbt)


def _edge_enc_body(ef, w1t, b1, g, be, w2t, b2, eh_o):
    x = jnp.dot(ef[...], w1t[...], preferred_element_type=jnp.float32) + b1[...]
    x = jax.nn.relu(_ln(x, g[...], be[...]))
    eh_o[...] = jnp.dot(x, w2t[...], preferred_element_type=jnp.float32) + b2[...]


def _edge_enc(ef, p):
    return pl.pallas_call(
        _edge_enc_body,
        grid=(E_PAD // BLK_E,),
        in_specs=[pl.BlockSpec((BLK_E, H), lambda i: (i, 0)),
                  _full(), _full((1, H)), _full((1, H)), _full((1, H)),
                  _full(), _full((1, H))],
        out_specs=pl.BlockSpec((BLK_E, H), lambda i: (i, 0)),
        out_shape=jax.ShapeDtypeStruct((E_PAD, H), jnp.float32),
    )(ef, p['ee_W1'].T, p['ee_b1'][None], p['ee_g'][None],
      p['ee_be'][None], p['ee_W2'].T, p['ee_b2'][None])


def _mlp_body(gs, gd, eh, w1et, b1, g, be, w2t, b2, gw, gb, m_o):
    m1 = (gs[...] + gd[...]
          + jnp.dot(eh[...], w1et[...], preferred_element_type=jnp.float32)
          + b1[...])
    m = jax.nn.relu(_ln(m1, g[...], be[...]))
    m = jnp.dot(m, w2t[...], preferred_element_type=jnp.float32) + b2[...]
    gate = jax.nn.sigmoid(
        jnp.sum(eh[...] * gw[...], axis=1, keepdims=True) + gb[...])
    m_o[...] = m * gate


def _edge_mlp(i, gs, gd, eh, p):
    return pl.pallas_call(
        _mlp_body,
        grid=(E_PAD // BLK_E,),
        in_specs=[pl.BlockSpec((BLK_E, H), lambda j: (j, 0)),
                  pl.BlockSpec((BLK_E, H), lambda j: (j, 0)),
                  pl.BlockSpec((BLK_E, H), lambda j: (j, 0)),
                  _full(), _full((1, H)), _full((1, H)), _full((1, H)),
                  _full(), _full((1, H)), _full((1, H)), _full((1, 1))],
        out_specs=pl.BlockSpec((BLK_E, H), lambda j: (j, 0)),
        out_shape=jax.ShapeDtypeStruct((E_PAD, H), jnp.float32),
    )(gs, gd, eh, p[f'mp{i}_W1'][:, 2 * H:3 * H].T, p[f'mp{i}_b1'][None],
      p[f'mp{i}_g'][None], p[f'mp{i}_be'][None],
      p[f'mp{i}_W2'].T, p[f'mp{i}_b2'][None],
      p[f'mp{i}_gW'], p[f'mp{i}_gb'][None])


def _gru_body(has_upd, aggp, nh, wit, bi, wht, bh, wat, wbt, ncwt, ncb,
              nh_o, a_o, b_o, upd_o=None):
    agg = aggp[0] + aggp[1]
    gi = jnp.dot(agg, wit[...], preferred_element_type=jnp.float32) + bi[...]
    gh = jnp.dot(nh[...], wht[...], preferred_element_type=jnp.float32) + bh[...]
    r = jax.nn.sigmoid(gi[:, :H] + gh[:, :H])
    z = jax.nn.sigmoid(gi[:, H:2 * H] + gh[:, H:2 * H])
    n = jnp.tanh(gi[:, 2 * H:] + r * gh[:, 2 * H:])
    nh_new = (1.0 - z) * n + z * nh[...]
    nh_o[...] = nh_new
    a_o[...] = jnp.dot(nh_new, wat[...], preferred_element_type=jnp.float32)
    b_o[...] = jnp.dot(nh_new, wbt[...], preferred_element_type=jnp.float32)
    if has_upd:
        upd_o[...] = jnp.dot(nh_new, ncwt[...],
                             preferred_element_type=jnp.float32) + ncb[...]


def _gru(i, aggp, nh, p, wat, wbt, last):
    f32 = jnp.float32
    n_out = 4 if last else 3
    blk = pl.BlockSpec((BLK_N, H), lambda j: (j, 0))
    return pl.pallas_call(
        functools.partial(_gru_body, last),
        grid=(N_PAD // BLK_N,),
        in_specs=[pl.BlockSpec((2, BLK_N, H), lambda j: (0, j, 0)), blk,
                  _full((H, 3 * H)), _full((1, 3 * H)),
                  _full((H, 3 * H)), _full((1, 3 * H)),
                  _full(), _full(), _full(), _full((1, H))],
        out_specs=[blk] * n_out,
        out_shape=[jax.ShapeDtypeStruct((N_PAD, H), f32)] * n_out,
    )(aggp, nh, p[f'mp{i}_Wi'].T, p[f'mp{i}_bi'][None],
      p[f'mp{i}_Wh'].T, p[f'mp{i}_bh'][None], wat, wbt,
      p['nc_W'].T, p['nc_b'][None])


def _scorer_body(gs, gd, b1, w2, b2, s_o):
    es = jax.nn.relu(gs[...] + gd[...] + b1[...])
    s_o[...] = jax.nn.sigmoid(
        jnp.sum(es * w2[...], axis=1, keepdims=True) + b2[...])


def _scorer(gs, gd, p):
    return pl.pallas_call(
        _scorer_body,
        grid=(E_PAD // BLK_E,),
        in_specs=[pl.BlockSpec((BLK_E, H), lambda j: (j, 0)),
                  pl.BlockSpec((BLK_E, H), lambda j: (j, 0)),
                  _full((1, H)), _full((1, H)), _full((1, 1))],
        out_specs=pl.BlockSpec((BLK_E, 1), lambda j: (j, 0)),
        out_shape=jax.ShapeDtypeStruct((E_PAD, 1), jnp.float32),
    )(gs, gd, p['ep_b1'][None], p['ep_W2'], p['ep_b2'][None])


def _gather2(a, b, srcr, dstr):
    f32 = jnp.float32

    @functools.partial(
        pl.kernel,
        out_type=[jax.ShapeDtypeStruct((E_PAD, H), f32)] * 2,
        mesh=_VMESH,
        scratch_types=[
            pltpu.VMEM((GNCH, GCH), jnp.int32),
            pltpu.VMEM((GNCH, GCH), jnp.int32),
        ] + [pltpu.VMEM((GCH, H), f32)] * 8
        + [pltpu.SemaphoreType.DMA] * 16,
    )
    def k(a_hbm, b_hbm, srcr_hbm, dstr_hbm, gs_hbm, gd_hbm,
          idxs_v, idxd_v, *rest):
        bufa = rest[0:4]
        bufb = rest[4:8]
        sa = rest[8:12]
        sb = rest[12:16]
        soa = rest[16:20]
        sob = rest[20:24]
        wid = lax.axis_index("s") * NC + lax.axis_index("c")
        base = wid * EB
        pltpu.sync_copy(srcr_hbm.at[wid], idxs_v)
        pltpu.sync_copy(dstr_hbm.at[wid], idxd_v)

        def issue_gather(j, c):
            pltpu.async_copy(a_hbm.at[idxs_v.at[c]], bufa[j], sa[j])
            pltpu.async_copy(b_hbm.at[idxd_v.at[c]], bufb[j], sb[j])

        def wait_gather(j):
            pltpu.make_async_copy(a_hbm.at[pl.ds(0, GCH)], bufa[j],
                                  sa[j]).wait()
            pltpu.make_async_copy(a_hbm.at[pl.ds(0, GCH)], bufb[j],
                                  sb[j]).wait()

        def issue_out(j, c):
            row = pl.ds(base + c * GCH, GCH)
            pltpu.async_copy(bufa[j], gs_hbm.at[row], soa[j])
            pltpu.async_copy(bufb[j], gd_hbm.at[row], sob[j])

        def wait_out(j):
            pltpu.make_async_copy(a_hbm.at[pl.ds(0, GCH)], bufa[j],
                                  soa[j]).wait()
            pltpu.make_async_copy(a_hbm.at[pl.ds(0, GCH)], bufb[j],
                                  sob[j]).wait()

        issue_gather(0, 0)
        issue_gather(1, 1)
        for c0 in (0, 1):
            wait_gather(c0)
            issue_out(c0, c0)
            issue_gather(c0 + 2, c0 + 2)

        @pl.loop(0, (GNCH - 4) // 4)
        def _(kk):
            for jj in range(4):
                c = kk * 4 + 2 + jj
                j = (2 + jj) % 4
                wait_gather(j)
                issue_out(j, c)
                wait_out(jj)
                issue_gather(jj, c + 2)

        for jj in range(4):
            c = GNCH - 2 + jj
            j = (2 + jj) % 4
            if jj < 2:
                wait_gather(j)
                issue_out(j, c)
            wait_out(j)

    return k(a, b, srcr, dstr)


def _scatter_add(m, dstr):
    f32 = jnp.float32

    @functools.partial(
        pl.kernel,
        out_type=jax.ShapeDtypeStruct((NC, N_PAD, H), f32),
        mesh=_VMESH,
        scratch_types=[
            pltpu.VMEM_SHARED((N_PAD, H), f32),
            pltpu.VMEM((CH, H), f32),
            pltpu.VMEM((CH, H), f32),
            pltpu.VMEM((NCH, CH), jnp.int32),
            pltpu.SemaphoreType.DMA,
            pltpu.SemaphoreType.DMA,
        ],
    )
    def k(m_hbm, dstr_hbm, out_hbm, acc_sh, m0, m1, idx_v, sm0, sm1):
        bufm = (m0, m1)
        sm = (sm0, sm1)
        cid = lax.axis_index("c")
        sid = lax.axis_index("s")
        wid = sid * NC + cid
        base = wid * EB

        @pl.loop(0, CH)
        def _(r):
            for c in range(0, H, 16):
                m0.at[r, pl.ds(c, 16)][...] = jnp.zeros((16,), f32)

        @pl.loop(0, NROWS // CH)
        def _(jj):
            pltpu.sync_copy(m0, acc_sh.at[pl.ds(sid * NROWS + jj * CH, CH)])

        plsc.subcore_barrier()
        pltpu.sync_copy(dstr_hbm.at[wid], idx_v)

        def issue_m(j, c):
            pltpu.async_copy(m_hbm.at[pl.ds(base + c * CH, CH)], bufm[j], sm[j])

        def wait_m(j):
            pltpu.make_async_copy(m_hbm.at[pl.ds(0, CH)], bufm[j], sm[j]).wait()

        for j in (0, 1):
            issue_m(j, j)

        @pl.loop(0, NCH // 2)
        def _(kk):
            for j in (0, 1):
                c = kk * 2 + j
                wait_m(j)
                pltpu.sync_copy(bufm[j], acc_sh.at[idx_v.at[c]], add=True)

                @pl.when(c + 2 < NCH)
                def _():
                    issue_m(j, c + 2)

        plsc.subcore_barrier()
        pltpu.sync_copy(acc_sh.at[pl.ds(sid * NROWS, NROWS)],
                        out_hbm.at[cid, pl.ds(sid * NROWS, NROWS)])

    return k(m, dstr)


def _pack_bf16(x):
    return jax.lax.bitcast_convert_type(
        x.reshape(x.shape[0], HW, 2), jnp.int32)


def _unpack_bf16(x):
    return jax.lax.bitcast_convert_type(
        x, jnp.bfloat16).reshape(x.shape[0], H)


@jax.jit
def _run(node_features, edge_indices, edge_features, p):
    nf = jnp.zeros((N_PAD, H), jnp.float32).at[:node_features.shape[0]].set(
        node_features)
    ef = jnp.zeros((E_PAD, H), jnp.float32).at[:edge_features.shape[0]].set(
        edge_features)
    e = edge_features.shape[0]
    n = node_features.shape[0]
    src = jnp.full((E_PAD,), 0, jnp.int32).at[:e].set(edge_indices[0])
    dst = jnp.full((E_PAD,), n, jnp.int32).at[:e].set(edge_indices[1])
    src_g = src.reshape(NW, GNCH, GCH)
    dst_g = dst.reshape(NW, GNCH, GCH)
    dst_s = dst.reshape(NW, NCH, CH)

    sw = lambda i: p[f'mp{i}_W1'][:, :H].T
    dw = lambda i: p[f'mp{i}_W1'][:, H:2 * H].T

    nh, a, b = _node_encoder(nf, p, sw(0), dw(0))
    eh = _edge_enc(ef, p)

    for i in range(3):
        gs, gd = _gather2(a, b, src_g, dst_g)
        m = _edge_mlp(i, gs, gd, eh, p)
        aggp = _scatter_add(m, dst_s)
        last = i == 2
        if last:
            nwat, nwbt = p['ep_W1'][:, :H].T, p['ep_W1'][:, H:].T
        else:
            nwat, nwbt = sw(i + 1), dw(i + 1)
        outs = _gru(i, aggp, nh, p, nwat, nwbt, last)
        nh, a, b = outs[0], outs[1], outs[2]
        if last:
            upd = outs[3]

    gs, gd = _gather2(a, b, src_g, dst_g)
    scores = _scorer(gs, gd, p)
    return upd[:n], scores[:e]


def kernel(node_features, node_boxes, edge_indices, edge_features, params):
    del node_boxes
    return _run(node_features, edge_indices, edge_features, params)


# speedup vs baseline: 1.0840x; 1.0574x over previous
import functools

import jax
import jax.numpy as jnp
from jax import lax
from jax.experimental import pallas as pl
from jax.experimental.pallas import tpu as pltpu
from jax.experimental.pallas import tpu_sc as plsc

H = 128
N_PAD = 10240
E_PAD = 327680
BLK_N = 1024
BLK_E = 4096

NC = 2
NS = 16
NW = NC * NS
EB = E_PAD // NW
CH = 128
NCH = EB // CH
NROWS = N_PAD // NS

_VMESH = plsc.VectorSubcoreMesh(core_axis_name="c", subcore_axis_name="s")


def _ln(x, g, b):
    m = jnp.mean(x, axis=-1, keepdims=True)
    v = jnp.mean((x - m) ** 2, axis=-1, keepdims=True)
    return (x - m) * jax.lax.rsqrt(v + 1e-5) * g + b


def _full(shape=(H, H)):
    return pl.BlockSpec(shape, lambda i: (0,) * len(shape))


def _node_encoder_body(nf, w1t, b1, g, be, w2t, b2, wat, wbt, nh_o, a_o, b_o):
    x = jnp.dot(nf[...], w1t[...], preferred_element_type=jnp.float32) + b1[...]
    x = jax.nn.relu(_ln(x, g[...], be[...]))
    nh = jnp.dot(x, w2t[...], preferred_element_type=jnp.float32) + b2[...]
    nh_o[...] = nh
    a_o[...] = jnp.dot(nh, wat[...], preferred_element_type=jnp.float32)
    b_o[...] = jnp.dot(nh, wbt[...], preferred_element_type=jnp.float32)


def _node_encoder(nf, p, wat, wbt):
    f32 = jnp.float32
    return pl.pallas_call(
        _node_encoder_body,
        grid=(N_PAD // BLK_N,),
        in_specs=[
            pl.BlockSpec((BLK_N, H), lambda i: (i, 0)),
            _full(), _full((1, H)), _full((1, H)), _full((1, H)),
            _full(), _full((1, H)), _full(), _full(),
        ],
        out_specs=[pl.BlockSpec((BLK_N, H), lambda i: (i, 0))] * 3,
        out_shape=[jax.ShapeDtypeStruct((N_PAD, H), f32)] * 3,
    )(nf, p['ne_W1'].T, p['ne_b1'][None], p['ne_g'][None], p['ne_be'][None],
      p['ne_W2'].T, p['ne_b2'][None], wat, wbt)


def _edge_enc_body(ef, w1t, b1, g, be, w2t, b2, eh_o):
    x = jnp.dot(ef[...], w1t[...], preferred_element_type=jnp.float32) + b1[...]
    x = jax.nn.relu(_ln(x, g[...], be[...]))
    eh_o[...] = jnp.dot(x, w2t[...], preferred_element_type=jnp.float32) + b2[...]


def _edge_enc(ef, p):
    return pl.pallas_call(
        _edge_enc_body,
        grid=(E_PAD // BLK_E,),
        in_specs=[pl.BlockSpec((BLK_E, H), lambda i: (i, 0)),
                  _full(), _full((1, H)), _full((1, H)), _full((1, H)),
                  _full(), _full((1, H))],
        out_specs=pl.BlockSpec((BLK_E, H), lambda i: (i, 0)),
        out_shape=jax.ShapeDtypeStruct((E_PAD, H), jnp.float32),
    )(ef, p['ee_W1'].T, p['ee_b1'][None], p['ee_g'][None],
      p['ee_be'][None], p['ee_W2'].T, p['ee_b2'][None])


def _mlp_body(gab, eh, w1et, b1, g, be, w2t, b2, gw, gb, m_o):
    m1 = (gab[...]
          + jnp.dot(eh[...], w1et[...], preferred_element_type=jnp.float32)
          + b1[...])
    m = jax.nn.relu(_ln(m1, g[...], be[...]))
    m = jnp.dot(m, w2t[...], preferred_element_type=jnp.float32) + b2[...]
    gate = jax.nn.sigmoid(
        jnp.sum(eh[...] * gw[...], axis=1, keepdims=True) + gb[...])
    m_o[...] = m * gate


def _edge_mlp(i, gab, eh, p):
    return pl.pallas_call(
        _mlp_body,
        grid=(E_PAD // BLK_E,),
        in_specs=[pl.BlockSpec((BLK_E, H), lambda j: (j, 0)),
                  pl.BlockSpec((BLK_E, H), lambda j: (j, 0)),
                  _full(), _full((1, H)), _full((1, H)), _full((1, H)),
                  _full(), _full((1, H)), _full((1, H)), _full((1, 1))],
        out_specs=pl.BlockSpec((BLK_E, H), lambda j: (j, 0)),
        out_shape=jax.ShapeDtypeStruct((E_PAD, H), jnp.float32),
    )(gab, eh, p[f'mp{i}_W1'][:, 2 * H:3 * H].T, p[f'mp{i}_b1'][None],
      p[f'mp{i}_g'][None], p[f'mp{i}_be'][None],
      p[f'mp{i}_W2'].T, p[f'mp{i}_b2'][None],
      p[f'mp{i}_gW'], p[f'mp{i}_gb'][None])


def _gru_body(has_upd, aggp, nh, wit, bi, wht, bh, wat, wbt, ncwt, ncb,
              nh_o, a_o, b_o, upd_o=None):
    agg = aggp[0] + aggp[1]
    gi = jnp.dot(agg, wit[...], preferred_element_type=jnp.float32) + bi[...]
    gh = jnp.dot(nh[...], wht[...], preferred_element_type=jnp.float32) + bh[...]
    r = jax.nn.sigmoid(gi[:, :H] + gh[:, :H])
    z = jax.nn.sigmoid(gi[:, H:2 * H] + gh[:, H:2 * H])
    n = jnp.tanh(gi[:, 2 * H:] + r * gh[:, 2 * H:])
    nh_new = (1.0 - z) * n + z * nh[...]
    nh_o[...] = nh_new
    a_o[...] = jnp.dot(nh_new, wat[...], preferred_element_type=jnp.float32)
    b_o[...] = jnp.dot(nh_new, wbt[...], preferred_element_type=jnp.float32)
    if has_upd:
        upd_o[...] = jnp.dot(nh_new, ncwt[...],
                             preferred_element_type=jnp.float32) + ncb[...]


def _gru(i, aggp, nh, p, wat, wbt, last):
    f32 = jnp.float32
    n_out = 4 if last else 3
    blk = pl.BlockSpec((BLK_N, H), lambda j: (j, 0))
    return pl.pallas_call(
        functools.partial(_gru_body, last),
        grid=(N_PAD // BLK_N,),
        in_specs=[pl.BlockSpec((2, BLK_N, H), lambda j: (0, j, 0)), blk,
                  _full((H, 3 * H)), _full((1, 3 * H)),
                  _full((H, 3 * H)), _full((1, 3 * H)),
                  _full(), _full(), _full(), _full((1, H))],
        out_specs=[blk] * n_out,
        out_shape=[jax.ShapeDtypeStruct((N_PAD, H), f32)] * n_out,
    )(aggp, nh, p[f'mp{i}_Wi'].T, p[f'mp{i}_bi'][None],
      p[f'mp{i}_Wh'].T, p[f'mp{i}_bh'][None], wat, wbt,
      p['nc_W'].T, p['nc_b'][None])


def _scorer_body(g2, b1, w2, b2, s_o):
    es = jax.nn.relu(g2[...] + b1[...])
    s_o[...] = jax.nn.sigmoid(
        jnp.sum(es * w2[...], axis=1, keepdims=True) + b2[...])


def _scorer(g2, p):
    return pl.pallas_call(
        _scorer_body,
        grid=(E_PAD // BLK_E,),
        in_specs=[pl.BlockSpec((BLK_E, H), lambda j: (j, 0)),
                  _full((1, H)), _full((1, H)), _full((1, 1))],
        out_specs=pl.BlockSpec((BLK_E, 1), lambda j: (j, 0)),
        out_shape=jax.ShapeDtypeStruct((E_PAD, 1), jnp.float32),
    )(g2, p['ep_b1'][None], p['ep_W2'], p['ep_b2'][None])


def _vadd(dst_ref, a_ref, b_ref):
    @pl.loop(0, CH)
    def _(r):
        for c in range(0, H, 16):
            s = (r, pl.ds(c, 16))
            dst_ref.at[*s][...] = a_ref.at[*s][...] + b_ref.at[*s][...]


def _gather_add(a, b, srcr, dstr):
    f32 = jnp.float32

    @functools.partial(
        pl.kernel,
        out_type=jax.ShapeDtypeStruct((E_PAD, H), f32),
        mesh=_VMESH,
        scratch_types=[
            pltpu.VMEM((NCH, CH), jnp.int32),
            pltpu.VMEM((NCH, CH), jnp.int32),
        ] + [pltpu.VMEM((CH, H), f32)] * 6
        + [pltpu.SemaphoreType.DMA] * 6,
    )
    def k(a_hbm, b_hbm, srcr_hbm, dstr_hbm, out_hbm,
          idxs_v, idxd_v, a0, a1, b0, b1, o0, o1,
          sa0, sa1, sb0, sb1, so0, so1):
        bufa = (a0, a1)
        bufb = (b0, b1)
        bufo = (o0, o1)
        sa = (sa0, sa1)
        sb = (sb0, sb1)
        so = (so0, so1)
        wid = lax.axis_index("s") * NC + lax.axis_index("c")
        base = wid * EB
        pltpu.sync_copy(srcr_hbm.at[wid], idxs_v)
        pltpu.sync_copy(dstr_hbm.at[wid], idxd_v)

        def issue_gather(j, c):
            pltpu.async_copy(a_hbm.at[idxs_v.at[c]], bufa[j], sa[j])
            pltpu.async_copy(b_hbm.at[idxd_v.at[c]], bufb[j], sb[j])

        def wait_gather(j):
            pltpu.make_async_copy(a_hbm.at[pl.ds(0, CH)], bufa[j], sa[j]).wait()
            pltpu.make_async_copy(a_hbm.at[pl.ds(0, CH)], bufb[j], sb[j]).wait()

        def issue_out(j, c):
            pltpu.async_copy(bufo[j], out_hbm.at[pl.ds(base + c * CH, CH)],
                             so[j])

        def wait_out(j):
            pltpu.make_async_copy(a_hbm.at[pl.ds(0, CH)], bufo[j], so[j]).wait()

        for j in (0, 1):
            issue_gather(j, j)
        for j in (0, 1):
            wait_gather(j)
            _vadd(bufo[j], bufa[j], bufb[j])
            issue_gather(j, j + 2)
            issue_out(j, j)

        @pl.loop(1, NCH // 2)
        def _(kk):
            for j in (0, 1):
                c = kk * 2 + j
                wait_gather(j)
                wait_out(j)
                _vadd(bufo[j], bufa[j], bufb[j])

                @pl.when(c + 2 < NCH)
                def _():
                    issue_gather(j, c + 2)

                issue_out(j, c)

        for j in (0, 1):
            wait_out(j)

    return k(a, b, srcr, dstr)


def _scatter_add(m, dstr):
    f32 = jnp.float32

    @functools.partial(
        pl.kernel,
        out_type=jax.ShapeDtypeStruct((NC, N_PAD, H), f32),
        mesh=_VMESH,
        scratch_types=[
            pltpu.VMEM_SHARED((N_PAD, H), f32),
            pltpu.VMEM((CH, H), f32),
            pltpu.VMEM((CH, H), f32),
            pltpu.VMEM((NCH, CH), jnp.int32),
            pltpu.SemaphoreType.DMA,
            pltpu.SemaphoreType.DMA,
        ],
    )
    def k(m_hbm, dstr_hbm, out_hbm, acc_sh, m0, m1, idx_v, sm0, sm1):
        bufm = (m0, m1)
        sm = (sm0, sm1)
        cid = lax.axis_index("c")
        sid = lax.axis_index("s")
        wid = sid * NC + cid
        base = wid * EB

        @pl.loop(0, CH)
        def _(r):
            for c in range(0, H, 16):
                m0.at[r, pl.ds(c, 16)][...] = jnp.zeros((16,), f32)

        @pl.loop(0, NROWS // CH)
        def _(jj):
            pltpu.sync_copy(m0, acc_sh.at[pl.ds(sid * NROWS + jj * CH, CH)])

        plsc.subcore_barrier()
        pltpu.sync_copy(dstr_hbm.at[wid], idx_v)

        def issue_m(j, c):
            pltpu.async_copy(m_hbm.at[pl.ds(base + c * CH, CH)], bufm[j], sm[j])

        def wait_m(j):
            pltpu.make_async_copy(m_hbm.at[pl.ds(0, CH)], bufm[j], sm[j]).wait()

        for j in (0, 1):
            issue_m(j, j)

        @pl.loop(0, NCH // 2)
        def _(kk):
            for j in (0, 1):
                c = kk * 2 + j
                wait_m(j)
                pltpu.sync_copy(bufm[j], acc_sh.at[idx_v.at[c]], add=True)

                @pl.when(c + 2 < NCH)
                def _():
                    issue_m(j, c + 2)

        plsc.subcore_barrier()
        pltpu.sync_copy(acc_sh.at[pl.ds(sid * NROWS, NROWS)],
                        out_hbm.at[cid, pl.ds(sid * NROWS, NROWS)])

    return k(m, dstr)


@jax.jit
def _run(node_features, edge_indices, edge_features, p):
    nf = jnp.zeros((N_PAD, H), jnp.float32).at[:node_features.shape[0]].set(
        node_features)
    ef = jnp.zeros((E_PAD, H), jnp.float32).at[:edge_features.shape[0]].set(
        edge_features)
    e = edge_features.shape[0]
    n = node_features.shape[0]
    src = jnp.full((E_PAD,), 0, jnp.int32).at[:e].set(edge_indices[0])
    dst = jnp.full((E_PAD,), n, jnp.int32).at[:e].set(edge_indices[1])
    src_r = src.reshape(NW, NCH, CH)
    dst_r = dst.reshape(NW, NCH, CH)

    sw = lambda i: p[f'mp{i}_W1'][:, :H].T
    dw = lambda i: p[f'mp{i}_W1'][:, H:2 * H].T

    nh, a, b = _node_encoder(nf, p, sw(0), dw(0))
    eh = _edge_enc(ef, p)

    for i in range(3):
        gab = _gather_add(a, b, src_r, dst_r)
        m = _edge_mlp(i, gab, eh, p)
        aggp = _scatter_add(m, dst_r)
        last = i == 2
        if last:
            nwat, nwbt = p['ep_W1'][:, :H].T, p['ep_W1'][:, H:].T
        else:
            nwat, nwbt = sw(i + 1), dw(i + 1)
        outs = _gru(i, aggp, nh, p, nwat, nwbt, last)
        nh, a, b = outs[0], outs[1], outs[2]
        if last:
            upd = outs[3]

    g2 = _gather_add(a, b, src_r, dst_r)
    scores = _scorer(g2, p)
    return upd[:n], scores[:e]


def kernel(node_features, node_boxes, edge_indices, edge_features, params):
    del node_boxes
    return _run(node_features, edge_indices, edge_features, params)


# speedup vs baseline: 1.1626x; 1.0725x over previous
import functools

import jax
import jax.numpy as jnp
from jax import lax
from jax.experimental import pallas as pl
from jax.experimental.pallas import tpu as pltpu
from jax.experimental.pallas import tpu_sc as plsc

H = 128
N_PAD = 10240
E_PAD = 327680
BLK_N = 1024
BLK_E = 4096

NC = 2
NS = 16
NW = NC * NS
EB = E_PAD // NW
CH = 128
NCH = EB // CH
GCH = 64
GNCH = EB // GCH
NROWS = N_PAD // NS

_VMESH = plsc.VectorSubcoreMesh(core_axis_name="c", subcore_axis_name="s")


def _ln(x, g, b):
    m = jnp.mean(x, axis=-1, keepdims=True)
    v = jnp.mean((x - m) ** 2, axis=-1, keepdims=True)
    return (x - m) * jax.lax.rsqrt(v + 1e-5) * g + b


def _full(shape=(H, H)):
    return pl.BlockSpec(shape, lambda i: (0,) * len(shape))


def _node_encoder_body(nf, w1t, b1, g, be, w2t, b2, wat, wbt, nh_o, a_o, b_o):
    x = jnp.dot(nf[...], w1t[...], preferred_element_type=jnp.float32) + b1[...]
    x = jax.nn.relu(_ln(x, g[...], be[...]))
    nh = jnp.dot(x, w2t[...], preferred_element_type=jnp.float32) + b2[...]
    nh_o[...] = nh
    a_o[...] = jnp.dot(nh, wat[...], preferred_element_type=jnp.float32)
    b_o[...] = jnp.dot(nh, wbt[...], preferred_element_type=jnp.float32)


def _node_encoder(nf, p, wat, wbt):
    f32 = jnp.float32
    return pl.pallas_call(
        _node_encoder_body,
        grid=(N_PAD // BLK_N,),
        in_specs=[
            pl.BlockSpec((BLK_N, H), lambda i: (i, 0)),
            _full(), _full((1, H)), _full((1, H)), _full((1, H)),
            _full(), _full((1, H)), _full(), _full(),
        ],
        out_specs=[pl.BlockSpec((BLK_N, H), lambda i: (i, 0))] * 3,
        out_shape=[jax.ShapeDtypeStruct((N_PAD, H), f32)] * 3,
    )(nf, p['ne_W1'].T, p['ne_b1'][None], p['ne_g'][None], p['ne_be'][None],
      p['ne_W2'].T, p['ne_b2'][None], wat, wbt)


def _edge_enc_body(ef, w1t, b1, g, be, w2t, b2, eh_o):
    x = jnp.dot(ef[...], w1t[...], preferred_element_type=jnp.float32) + b1[...]
    x = jax.nn.relu(_ln(x, g[...], be[...]))
    eh_o[...] = jnp.dot(x, w2t[...], preferred_element_type=jnp.float32) + b2[...]


def _edge_enc(ef, p):
    return pl.pallas_call(
        _edge_enc_body,
        grid=(E_PAD // BLK_E,),
        in_specs=[pl.BlockSpec((BLK_E, H), lambda i: (i, 0)),
                  _full(), _full((1, H)), _full((1, H)), _full((1, H)),
                  _full(), _full((1, H))],
        out_specs=pl.BlockSpec((BLK_E, H), lambda i: (i, 0)),
        out_shape=jax.ShapeDtypeStruct((E_PAD, H), jnp.float32),
    )(ef, p['ee_W1'].T, p['ee_b1'][None], p['ee_g'][None],
      p['ee_be'][None], p['ee_W2'].T, p['ee_b2'][None])


def _mlp_body(gab, eh, w1et, b1, g, be, w2t, b2, gw, gb, m_o):
    m1 = (gab[...]
          + jnp.dot(eh[...], w1et[...], preferred_element_type=jnp.float32)
          + b1[...])
    m = jax.nn.relu(_ln(m1, g[...], be[...]))
    m = jnp.dot(m, w2t[...], preferred_element_type=jnp.float32) + b2[...]
    gate = jax.nn.sigmoid(
        jnp.sum(eh[...] * gw[...], axis=1, keepdims=True) + gb[...])
    m_o[...] = m * gate


def _edge_mlp(i, gab, eh, p):
    return pl.pallas_call(
        _mlp_body,
        grid=(E_PAD // BLK_E,),
        in_specs=[pl.BlockSpec((BLK_E, H), lambda j: (j, 0)),
                  pl.BlockSpec((BLK_E, H), lambda j: (j, 0)),
                  _full(), _full((1, H)), _full((1, H)), _full((1, H)),
                  _full(), _full((1, H)), _full((1, H)), _full((1, 1))],
        out_specs=pl.BlockSpec((BLK_E, H), lambda j: (j, 0)),
        out_shape=jax.ShapeDtypeStruct((E_PAD, H), jnp.float32),
    )(gab, eh, p[f'mp{i}_W1'][:, 2 * H:3 * H].T, p[f'mp{i}_b1'][None],
      p[f'mp{i}_g'][None], p[f'mp{i}_be'][None],
      p[f'mp{i}_W2'].T, p[f'mp{i}_b2'][None],
      p[f'mp{i}_gW'], p[f'mp{i}_gb'][None])


def _gru_body(has_upd, aggp, nh, wit, bi, wht, bh, wat, wbt, ncwt, ncb,
              nh_o, a_o, b_o, upd_o=None):
    agg = aggp[0] + aggp[1]
    gi = jnp.dot(agg, wit[...], preferred_element_type=jnp.float32) + bi[...]
    gh = jnp.dot(nh[...], wht[...], preferred_element_type=jnp.float32) + bh[...]
    r = jax.nn.sigmoid(gi[:, :H] + gh[:, :H])
    z = jax.nn.sigmoid(gi[:, H:2 * H] + gh[:, H:2 * H])
    n = jnp.tanh(gi[:, 2 * H:] + r * gh[:, 2 * H:])
    nh_new = (1.0 - z) * n + z * nh[...]
    nh_o[...] = nh_new
    a_o[...] = jnp.dot(nh_new, wat[...], preferred_element_type=jnp.float32)
    b_o[...] = jnp.dot(nh_new, wbt[...], preferred_element_type=jnp.float32)
    if has_upd:
        upd_o[...] = jnp.dot(nh_new, ncwt[...],
                             preferred_element_type=jnp.float32) + ncb[...]


def _gru(i, aggp, nh, p, wat, wbt, last):
    f32 = jnp.float32
    n_out = 4 if last else 3
    blk = pl.BlockSpec((BLK_N, H), lambda j: (j, 0))
    return pl.pallas_call(
        functools.partial(_gru_body, last),
        grid=(N_PAD // BLK_N,),
        in_specs=[pl.BlockSpec((2, BLK_N, H), lambda j: (0, j, 0)), blk,
                  _full((H, 3 * H)), _full((1, 3 * H)),
                  _full((H, 3 * H)), _full((1, 3 * H)),
                  _full(), _full(), _full(), _full((1, H))],
        out_specs=[blk] * n_out,
        out_shape=[jax.ShapeDtypeStruct((N_PAD, H), f32)] * n_out,
    )(aggp, nh, p[f'mp{i}_Wi'].T, p[f'mp{i}_bi'][None],
      p[f'mp{i}_Wh'].T, p[f'mp{i}_bh'][None], wat, wbt,
      p['nc_W'].T, p['nc_b'][None])


def _scorer_body(g2, b1, w2, b2, s_o):
    es = jax.nn.relu(g2[...] + b1[...])
    s_o[...] = jax.nn.sigmoid(
        jnp.sum(es * w2[...], axis=1, keepdims=True) + b2[...])


def _scorer(g2, p):
    return pl.pallas_call(
        _scorer_body,
        grid=(E_PAD // BLK_E,),
        in_specs=[pl.BlockSpec((BLK_E, H), lambda j: (j, 0)),
                  _full((1, H)), _full((1, H)), _full((1, 1))],
        out_specs=pl.BlockSpec((BLK_E, 1), lambda j: (j, 0)),
        out_shape=jax.ShapeDtypeStruct((E_PAD, 1), jnp.float32),
    )(g2, p['ep_b1'][None], p['ep_W2'], p['ep_b2'][None])


def _vadd(dst_ref, a_ref, b_ref):
    @pl.loop(0, GCH)
    def _(r):
        for c in range(0, H, 16):
            s = (r, pl.ds(c, 16))
            dst_ref.at[*s][...] = a_ref.at[*s][...] + b_ref.at[*s][...]


def _gather_add(a, b, srcr, dstr):
    f32 = jnp.float32

    @functools.partial(
        pl.kernel,
        out_type=jax.ShapeDtypeStruct((E_PAD, H), f32),
        mesh=_VMESH,
        scratch_types=[
            pltpu.VMEM((GNCH, GCH), jnp.int32),
            pltpu.VMEM((GNCH, GCH), jnp.int32),
        ] + [pltpu.VMEM((GCH, H), f32)] * 8
        + [pltpu.SemaphoreType.DMA] * 8,
    )
    def k(a_hbm, b_hbm, srcr_hbm, dstr_hbm, out_hbm,
          idxs_v, idxd_v, *rest):
        bufa = rest[0:3]
        bufb = rest[3:6]
        bufo = rest[6:8]
        sa = rest[8:11]
        sb = rest[11:14]
        so = rest[14:16]
        wid = lax.axis_index("s") * NC + lax.axis_index("c")
        base = wid * EB
        pltpu.sync_copy(srcr_hbm.at[wid], idxs_v)
        pltpu.sync_copy(dstr_hbm.at[wid], idxd_v)

        def issue_gather(j, c):
            pltpu.async_copy(a_hbm.at[idxs_v.at[c]], bufa[j], sa[j])
            pltpu.async_copy(b_hbm.at[idxd_v.at[c]], bufb[j], sb[j])

        def wait_gather(j):
            pltpu.make_async_copy(a_hbm.at[pl.ds(0, GCH)], bufa[j],
                                  sa[j]).wait()
            pltpu.make_async_copy(a_hbm.at[pl.ds(0, GCH)], bufb[j],
                                  sb[j]).wait()

        def issue_out(j, c):
            pltpu.async_copy(bufo[j], out_hbm.at[pl.ds(base + c * GCH, GCH)],
                             so[j])

        def wait_out(j):
            pltpu.make_async_copy(a_hbm.at[pl.ds(0, GCH)], bufo[j],
                                  so[j]).wait()

        for q in (0, 1, 2):
            issue_gather(q, q)
        for c0 in (0, 1):
            wait_gather(c0)
            _vadd(bufo[c0], bufa[c0], bufb[c0])
            issue_gather(c0, c0 + 3)
            issue_out(c0, c0)

        @pl.loop(0, (GNCH - 4) // 6)
        def _(kk):
            for jj in range(6):
                c = kk * 6 + 2 + jj
                j3 = (2 + jj) % 3
                j2 = jj % 2
                wait_gather(j3)
                wait_out(j2)
                _vadd(bufo[j2], bufa[j3], bufb[j3])

                @pl.when(c + 3 < GNCH)
                def _():
                    issue_gather(j3, c + 3)

                issue_out(j2, c)

        for jj in (0, 1):
            c = GNCH - 2 + jj
            j3 = c % 3
            j2 = c % 2
            wait_gather(j3)
            wait_out(j2)
            _vadd(bufo[j2], bufa[j3], bufb[j3])
            issue_out(j2, c)

        for j in (0, 1):
            wait_out(j)

    return k(a, b, srcr, dstr)


def _scatter_add(m, dstr):
    f32 = jnp.float32

    @functools.partial(
        pl.kernel,
        out_type=jax.ShapeDtypeStruct((NC, N_PAD, H), f32),
        mesh=_VMESH,
        scratch_types=[
            pltpu.VMEM_SHARED((N_PAD, H), f32),
            pltpu.VMEM((CH, H), f32),
            pltpu.VMEM((CH, H), f32),
            pltpu.VMEM((NCH, CH), jnp.int32),
            pltpu.SemaphoreType.DMA,
            pltpu.SemaphoreType.DMA,
        ],
    )
    def k(m_hbm, dstr_hbm, out_hbm, acc_sh, m0, m1, idx_v, sm0, sm1):
        bufm = (m0, m1)
        sm = (sm0, sm1)
        cid = lax.axis_index("c")
        sid = lax.axis_index("s")
        wid = sid * NC + cid
        base = wid * EB

        @pl.loop(0, CH)
        def _(r):
            for c in range(0, H, 16):
                m0.at[r, pl.ds(c, 16)][...] = jnp.zeros((16,), f32)

        @pl.loop(0, NROWS // CH)
        def _(jj):
            pltpu.sync_copy(m0, acc_sh.at[pl.ds(sid * NROWS + jj * CH, CH)])

        plsc.subcore_barrier()
        pltpu.sync_copy(dstr_hbm.at[wid], idx_v)

        def issue_m(j, c):
            pltpu.async_copy(m_hbm.at[pl.ds(base + c * CH, CH)], bufm[j], sm[j])

        def wait_m(j):
            pltpu.make_async_copy(m_hbm.at[pl.ds(0, CH)], bufm[j], sm[j]).wait()

        for j in (0, 1):
            issue_m(j, j)

        @pl.loop(0, NCH // 2)
        def _(kk):
            for j in (0, 1):
                c = kk * 2 + j
                wait_m(j)
                pltpu.sync_copy(bufm[j], acc_sh.at[idx_v.at[c]], add=True)

                @pl.when(c + 2 < NCH)
                def _():
                    issue_m(j, c + 2)

        plsc.subcore_barrier()
        pltpu.sync_copy(acc_sh.at[pl.ds(sid * NROWS, NROWS)],
                        out_hbm.at[cid, pl.ds(sid * NROWS, NROWS)])

    return k(m, dstr)


@jax.jit
def _run(node_features, edge_indices, edge_features, p):
    nf = jnp.zeros((N_PAD, H), jnp.float32).at[:node_features.shape[0]].set(
        node_features)
    ef = jnp.zeros((E_PAD, H), jnp.float32).at[:edge_features.shape[0]].set(
        edge_features)
    e = edge_features.shape[0]
    n = node_features.shape[0]
    src = jnp.full((E_PAD,), 0, jnp.int32).at[:e].set(edge_indices[0])
    dst = jnp.full((E_PAD,), n, jnp.int32).at[:e].set(edge_indices[1])
    src_g = src.reshape(NW, GNCH, GCH)
    dst_g = dst.reshape(NW, GNCH, GCH)
    dst_s = dst.reshape(NW, NCH, CH)

    sw = lambda i: p[f'mp{i}_W1'][:, :H].T
    dw = lambda i: p[f'mp{i}_W1'][:, H:2 * H].T

    nh, a, b = _node_encoder(nf, p, sw(0), dw(0))
    eh = _edge_enc(ef, p)

    for i in range(3):
        gab = _gather_add(a, b, src_g, dst_g)
        m = _edge_mlp(i, gab, eh, p)
        aggp = _scatter_add(m, dst_s)
        last = i == 2
        if last:
            nwat, nwbt = p['ep_W1'][:, :H].T, p['ep_W1'][:, H:].T
        else:
            nwat, nwbt = sw(i + 1), dw(i + 1)
        outs = _gru(i, aggp, nh, p, nwat, nwbt, last)
        nh, a, b = outs[0], outs[1], outs[2]
        if last:
            upd = outs[3]

    g2 = _gather_add(a, b, src_g, dst_g)
    scores = _scorer(g2, p)
    return upd[:n], scores[:e]


def kernel(node_features, node_boxes, edge_indices, edge_features, params):
    del node_boxes
    return _run(node_features, edge_indices, edge_features, params)
